# baseline (device time: 204212 ns/iter reference)
import jax
import jax.numpy as jnp
from jax import lax
from jax.experimental import pallas as pl
from jax.experimental.pallas import tpu as pltpu

N_DEV = 16
N_HOPS = 8

RING = (1, 5, 9, 13, 14, 10, 6, 2, 3, 7, 11, 15, 12, 8, 4, 0)
INV = tuple(RING.index(i) for i in range(N_DEV))


def _lookup(table, idx):
    v = jnp.int32(table[0])
    for j in range(1, len(table)):
        v = jnp.where(idx == j, jnp.int32(table[j]), v)
    return v


def kernel(x, w_mat):
    m_per, k = x.shape
    n_per = w_mat.shape[1]
    n_half = n_per // 2

    def body(x_ref, w_ref, out_ref, xb_ref, w_comm, blocks, blocks_in,
             ssr, rsr, ssl, rsl, a2a_ss, a2a_rs):
        me = lax.axis_index("i")
        r = _lookup(INV, me)
        right = _lookup(RING, (r + 1) % N_DEV)
        left = _lookup(RING, (r - 1) % N_DEV)
        my_rows = me * m_per

        xb_ref[...] = x_ref[...].astype(jnp.bfloat16)
        wb = w_ref[...].astype(jnp.bfloat16)
        w_comm[2 * r] = wb[:, :n_half]
        w_comm[2 * r + 1] = wb[:, n_half:]

        barrier_sem = pltpu.get_barrier_semaphore()
        for nbr in (left, right):
            pl.semaphore_signal(
                barrier_sem, inc=1,
                device_id=(nbr,), device_id_type=pl.DeviceIdType.MESH,
            )
        pl.semaphore_wait(barrier_sem, 2)

        def make_ring_rdma(hs, sems_s, sems_r, dev):
            return pltpu.make_async_remote_copy(
                src_ref=w_comm.at[hs],
                dst_ref=w_comm.at[hs],
                send_sem=sems_s.at[hs],
                recv_sem=sems_r.at[hs],
                device_id=(dev,),
                device_id_type=pl.DeviceIdType.MESH,
            )

        pending_sends = []

        def ring_send(hs, sems_s, sems_r, dev):
            rdma = make_ring_rdma(hs, sems_s, sems_r, dev)
            rdma.start()
            pending_sends.append(rdma)

        def block_rdma(s, dst_rows):
            del dst_rows
            return pltpu.make_async_remote_copy(
                src_ref=blocks.at[s],
                dst_ref=blocks_in.at[r],
                send_sem=a2a_ss.at[s],
                recv_sem=a2a_rs.at[r],
                device_id=(_lookup(RING, s),),
                device_id_type=pl.DeviceIdType.MESH,
            )

        def compute_and_push(s):
            for j in (0, 1):
                y = jnp.dot(
                    xb_ref[...], w_comm[2 * s + j],
                    preferred_element_type=jnp.float32,
                )
                blocks[s, :, j * n_half:(j + 1) * n_half] = jnp.maximum(y, 0.0)
            rdma = block_rdma(s, my_rows)
            rdma.start()
            pending_sends.append(rdma)

        ring_send(2 * r, ssr, rsr, right)
        ring_send(2 * r + 1, ssl, rsl, left)
        ring_send(2 * r + 1, ssr, rsr, right)
        ring_send(2 * r, ssl, rsl, left)

        for j in (0, 1):
            y = jnp.dot(
                xb_ref[...], w_comm[2 * r + j],
                preferred_element_type=jnp.float32,
            )
            out_ref[pl.ds(my_rows, m_per), pl.ds(j * n_half, n_half)] = (
                jnp.maximum(y, 0.0)
            )

        for h in range(1, N_HOPS + 1):
            rs = (r - h) % N_DEV
            ls = (r + h) % N_DEV

            make_ring_rdma(2 * rs, ssr, rsr, left).wait_recv()
            if h < N_HOPS:
                ring_send(2 * rs, ssr, rsr, right)
            make_ring_rdma(2 * ls + 1, ssl, rsl, right).wait_recv()
            if h < N_HOPS:
                ring_send(2 * ls + 1, ssl, rsl, left)

            if h < N_HOPS:
                make_ring_rdma(2 * rs + 1, ssr, rsr, left).wait_recv()
                if h < N_HOPS - 1:
                    ring_send(2 * rs + 1, ssr, rsr, right)
                make_ring_rdma(2 * ls, ssl, rsl, right).wait_recv()
                if h < N_HOPS - 1:
                    ring_send(2 * ls, ssl, rsl, left)

                compute_and_push(rs)
                compute_and_push(ls)
            else:
                compute_and_push(rs)

        for s_off in range(1, N_DEV):
            s = (r + s_off) % N_DEV
            pltpu.make_async_remote_copy(
                src_ref=blocks.at[s],
                dst_ref=blocks_in.at[s],
                send_sem=a2a_ss.at[s],
                recv_sem=a2a_rs.at[s],
                device_id=(me,),
                device_id_type=pl.DeviceIdType.MESH,
            ).wait_recv()
            rows = _lookup(RING, s) * m_per
            out_ref[pl.ds(rows, m_per), :] = blocks_in[s]

        for rdma in pending_sends:
            rdma.wait_send()

    return pl.pallas_call(
        body,
        out_shape=jax.ShapeDtypeStruct((N_DEV * m_per, n_per), jnp.float32),
        in_specs=[
            pl.BlockSpec(memory_space=pltpu.VMEM),
            pl.BlockSpec(memory_space=pltpu.VMEM),
        ],
        out_specs=pl.BlockSpec(memory_space=pltpu.VMEM),
        scratch_shapes=[
            pltpu.VMEM((m_per, k), jnp.bfloat16),
            pltpu.VMEM((2 * N_DEV, k, n_half), jnp.bfloat16),
            pltpu.VMEM((N_DEV, m_per, n_per), jnp.float32),
            pltpu.VMEM((N_DEV, m_per, n_per), jnp.float32),
            pltpu.SemaphoreType.DMA((2 * N_DEV,)),
            pltpu.SemaphoreType.DMA((2 * N_DEV,)),
            pltpu.SemaphoreType.DMA((2 * N_DEV,)),
            pltpu.SemaphoreType.DMA((2 * N_DEV,)),
            pltpu.SemaphoreType.DMA((N_DEV,)),
            pltpu.SemaphoreType.DMA((N_DEV,)),
        ],
        compiler_params=pltpu.CompilerParams(
            collective_id=0, vmem_limit_bytes=100 * 1024 * 1024
        ),
    )(x, w_mat)


# device time: 110405 ns/iter; 1.8497x vs baseline; 1.8497x over previous
import jax
import jax.numpy as jnp
from jax import lax
from jax.experimental import pallas as pl
from jax.experimental.pallas import tpu as pltpu

N_DEV = 16
N_HOPS = 8

RING = (1, 5, 9, 13, 14, 10, 6, 2, 3, 7, 11, 15, 12, 8, 4, 0)
INV = tuple(RING.index(i) for i in range(N_DEV))


def _lookup(table, idx):
    v = jnp.int32(table[0])
    for j in range(1, len(table)):
        v = jnp.where(idx == j, jnp.int32(table[j]), v)
    return v


def kernel(x, w_mat):
    m_per, k = x.shape
    n_per = w_mat.shape[1]
    k_half = k // 2

    def body(x_ref, w_ref, out_ref, xb_ref, w_comm, blocks, blocks_in,
             ssr, rsr, ssl, rsl, a2a_ss, a2a_rs):
        me = lax.axis_index("i")
        r = _lookup(INV, me)
        right = _lookup(RING, (r + 1) % N_DEV)
        left = _lookup(RING, (r - 1) % N_DEV)
        my_rows = me * m_per

        xb_ref[...] = x_ref[...].astype(jnp.bfloat16)
        wb = w_ref[...].astype(jnp.bfloat16)
        w_comm[2 * r] = wb[:k_half, :]
        w_comm[2 * r + 1] = wb[k_half:, :]

        barrier_sem = pltpu.get_barrier_semaphore()
        for nbr in (left, right):
            pl.semaphore_signal(
                barrier_sem, inc=1,
                device_id=(nbr,), device_id_type=pl.DeviceIdType.MESH,
            )
        pl.semaphore_wait(barrier_sem, 2)

        def make_ring_rdma(hs, sems_s, sems_r, dev):
            return pltpu.make_async_remote_copy(
                src_ref=w_comm.at[hs],
                dst_ref=w_comm.at[hs],
                send_sem=sems_s.at[hs],
                recv_sem=sems_r.at[hs],
                device_id=(dev,),
                device_id_type=pl.DeviceIdType.MESH,
            )

        pending_sends = []

        def ring_send(hs, sems_s, sems_r, dev):
            rdma = make_ring_rdma(hs, sems_s, sems_r, dev)
            rdma.start()
            pending_sends.append(rdma)

        def block_rdma(s, dst_rows):
            del dst_rows
            return pltpu.make_async_remote_copy(
                src_ref=blocks.at[s],
                dst_ref=blocks_in.at[r],
                send_sem=a2a_ss.at[s],
                recv_sem=a2a_rs.at[r],
                device_id=(_lookup(RING, s),),
                device_id_type=pl.DeviceIdType.MESH,
            )

        def full_gemm(s):
            y0 = jnp.dot(
                xb_ref[:, :k_half], w_comm[2 * s],
                preferred_element_type=jnp.float32,
            )
            y1 = jnp.dot(
                xb_ref[:, k_half:], w_comm[2 * s + 1],
                preferred_element_type=jnp.float32,
            )
            return jnp.maximum(y0 + y1, 0.0)

        def compute_and_push(s):
            blocks[s] = full_gemm(s).astype(jnp.bfloat16)
            rdma = block_rdma(s, my_rows)
            rdma.start()
            pending_sends.append(rdma)

        ring_send(2 * r, ssr, rsr, right)
        ring_send(2 * r + 1, ssl, rsl, left)
        ring_send(2 * r + 1, ssr, rsr, right)
        ring_send(2 * r, ssl, rsl, left)

        out_ref[pl.ds(my_rows, m_per), :] = full_gemm(r)

        for h in range(1, N_HOPS + 1):
            rs = (r - h) % N_DEV
            ls = (r + h) % N_DEV

            make_ring_rdma(2 * rs, ssr, rsr, left).wait_recv()
            if h < N_HOPS:
                ring_send(2 * rs, ssr, rsr, right)
            make_ring_rdma(2 * ls + 1, ssl, rsl, right).wait_recv()
            if h < N_HOPS:
                ring_send(2 * ls + 1, ssl, rsl, left)

            if h < N_HOPS:
                make_ring_rdma(2 * rs + 1, ssr, rsr, left).wait_recv()
                if h < N_HOPS - 1:
                    ring_send(2 * rs + 1, ssr, rsr, right)
                make_ring_rdma(2 * ls, ssl, rsl, right).wait_recv()
                if h < N_HOPS - 1:
                    ring_send(2 * ls, ssl, rsl, left)

                compute_and_push(rs)
                compute_and_push(ls)
            else:
                compute_and_push(rs)

        for s_off in range(1, N_DEV):
            s = (r + s_off) % N_DEV
            pltpu.make_async_remote_copy(
                src_ref=blocks.at[s],
                dst_ref=blocks_in.at[s],
                send_sem=a2a_ss.at[s],
                recv_sem=a2a_rs.at[s],
                device_id=(me,),
                device_id_type=pl.DeviceIdType.MESH,
            ).wait_recv()
            rows = _lookup(RING, s) * m_per
            out_ref[pl.ds(rows, m_per), :] = blocks_in[s].astype(jnp.float32)

        for rdma in pending_sends:
            rdma.wait_send()

    return pl.pallas_call(
        body,
        out_shape=jax.ShapeDtypeStruct((N_DEV * m_per, n_per), jnp.float32),
        in_specs=[
            pl.BlockSpec(memory_space=pltpu.VMEM),
            pl.BlockSpec(memory_space=pltpu.VMEM),
        ],
        out_specs=pl.BlockSpec(memory_space=pltpu.VMEM),
        scratch_shapes=[
            pltpu.VMEM((m_per, k), jnp.bfloat16),
            pltpu.VMEM((2 * N_DEV, k_half, n_per), jnp.bfloat16),
            pltpu.VMEM((N_DEV, m_per, n_per), jnp.bfloat16),
            pltpu.VMEM((N_DEV, m_per, n_per), jnp.bfloat16),
            pltpu.SemaphoreType.DMA((2 * N_DEV,)),
            pltpu.SemaphoreType.DMA((2 * N_DEV,)),
            pltpu.SemaphoreType.DMA((2 * N_DEV,)),
            pltpu.SemaphoreType.DMA((2 * N_DEV,)),
            pltpu.SemaphoreType.DMA((N_DEV,)),
            pltpu.SemaphoreType.DMA((N_DEV,)),
        ],
        compiler_params=pltpu.CompilerParams(
            collective_id=0, vmem_limit_bytes=100 * 1024 * 1024
        ),
    )(x, w_mat)
